# baseline (speedup 1.0000x reference)
import jax
import jax.numpy as jnp
from jax.experimental import pallas as pl

N_GRAPHS = 256


def _copy_body(x_ref, o_ref):
    o_ref[...] = x_ref[...]


def _gcn_conv(x, edge_index, W, b):
    n = x.shape[0]
    loop = jnp.arange(n, dtype=edge_index.dtype)
    src = jnp.concatenate([edge_index[0], loop])
    dst = jnp.concatenate([edge_index[1], loop])
    deg = jnp.zeros((n,), dtype=x.dtype).at[dst].add(1.0)
    dinv = jnp.where(deg > 0, 1.0 / jnp.sqrt(deg), 0.0)
    norm = dinv[src] * dinv[dst]
    h = x @ W
    msg = jnp.take(h, src, axis=0) * norm[:, None]
    out = jnp.zeros((n, W.shape[1]), dtype=x.dtype).at[dst].add(msg)
    return out + b


def _bn(h, gamma, beta):
    mu = jnp.mean(h, axis=0)
    var = jnp.var(h, axis=0)
    return gamma * (h - mu) / jnp.sqrt(var + 1e-5) + beta


def kernel(x, edge_index, batch, W1, b1, g1, be1, W2, b2, g2, be2, W3, b3):
    h1 = jax.nn.relu(_bn(_gcn_conv(x, edge_index, W1, b1), g1, be1))
    h2 = jax.nn.relu(_bn(_gcn_conv(h1, edge_index, W2, b2), g2, be2))
    h3 = _gcn_conv(h2, edge_index, W3, b3)
    counts = jax.ops.segment_sum(jnp.ones((h3.shape[0],), dtype=h3.dtype), batch, num_segments=N_GRAPHS)
    mean_pool = jax.ops.segment_sum(h3, batch, num_segments=N_GRAPHS) / jnp.maximum(counts, 1.0)[:, None]
    max_pool = jax.ops.segment_max(h3, batch, num_segments=N_GRAPHS)
    max_pool = jnp.where(counts[:, None] > 0, max_pool, 0.0)
    out = mean_pool + max_pool
    return pl.pallas_call(
        _copy_body,
        out_shape=jax.ShapeDtypeStruct(out.shape, out.dtype),
    )(out)


# speedup vs baseline: 11.9960x; 11.9960x over previous
import functools

import jax
import jax.numpy as jnp
from jax import lax
from jax.experimental import pallas as pl
from jax.experimental.pallas import tpu as pltpu
from jax.experimental.pallas import tpu_sc as plsc

N = 10000
E = 320000
G = 256
D_IN = 128
D_H = 256
D_OUT = 384

NC = 2
NS = 16
L = 16
NW = NC * NS

CH = 128
STRIPE = 624
ZROWS = 104
TAIL = N - NS * STRIPE
CHP = 64
CHW = 72
GPAD = 384
NEG = -3.0e38

_mesh = plsc.VectorSubcoreMesh(core_axis_name="c", subcore_axis_name="s")
_sc_params = pltpu.CompilerParams(needs_layout_passes=False)


def _deg_body(dst_hbm, out_hbm, idx_v, hist_v):
    c = lax.axis_index("c")
    s = lax.axis_index("s")
    wid = s * NC + c
    epw = E // NW
    zv = jnp.zeros((L,), jnp.float32)

    def zero(i, _):
        hist_v[pl.ds(i * L, L)] = zv
        return 0
    lax.fori_loop(0, N // L, zero, 0)

    pltpu.sync_copy(dst_hbm.at[pl.ds(wid * epw, epw)], idx_v)
    ones = jnp.ones((L,), jnp.float32)

    def body(i, _):
        iv = idx_v[pl.ds(i * L, L)]
        plsc.addupdate_scatter(hist_v, [iv], ones)
        return 0
    lax.fori_loop(0, epw // L, body, 0)

    pltpu.sync_copy(hist_v, out_hbm.at[wid])


_deg_call = functools.partial(
    pl.kernel,
    out_type=jax.ShapeDtypeStruct((NW, N), jnp.float32),
    mesh=_mesh,
    compiler_params=_sc_params,
    scratch_types=[
        pltpu.VMEM((E // NW,), jnp.int32),
        pltpu.VMEM((N,), jnp.float32),
    ],
)(_deg_body)


def _make_agg(d2, edge_split):
    def body(hs_hbm, src_hbm, dst_hbm, out_hbm, gidx, didx, rows, zbuf, S, sem):
        c = lax.axis_index("c")
        s = lax.axis_index("s")
        zv = jnp.zeros((L,), jnp.float32)

        def zb(r, _):
            for t in range(d2 // L):
                zbuf[r, pl.ds(t * L, L)] = zv
            return 0
        lax.fori_loop(0, ZROWS, zb, 0)
        stripe0 = pl.multiple_of(s * STRIPE, 8)
        for k in range(STRIPE // ZROWS):
            pltpu.sync_copy(zbuf, S.at[pl.ds(stripe0 + k * ZROWS, ZROWS)])

        @pl.when(s == 0)
        def _():
            pltpu.sync_copy(zbuf.at[pl.ds(0, TAIL)], S.at[pl.ds(NS * STRIPE, TAIL)])
        plsc.subcore_barrier()

        nrows = E // CH
        if edge_split:
            half = nrows // 2
            base_q, base_r = half // NS, half % NS
            row0 = c * half + base_q * s + jnp.minimum(s, base_r)
        else:
            base_q, base_r = nrows // NS, nrows % NS
            row0 = base_q * s + jnp.minimum(s, base_r)
        nch = base_q + (s < base_r).astype(jnp.int32)
        offv = jnp.broadcast_to(c * N, (L,)).astype(jnp.int32)

        def chunk(j, _):
            base = (row0 + j) * CH
            pltpu.sync_copy(src_hbm.at[pl.ds(base, CH)], gidx)
            if not edge_split:
                for t in range(CH // L):
                    gidx[pl.ds(t * L, L)] = gidx[pl.ds(t * L, L)] + offv
            pltpu.async_copy(hs_hbm.at[gidx], rows, sem).wait()
            pltpu.sync_copy(dst_hbm.at[pl.ds(base, CH)], didx)
            pltpu.sync_copy(rows, S.at[didx], add=True)
            return 0
        lax.fori_loop(0, nch, chunk, 0)
        plsc.subcore_barrier()

        ob = pl.multiple_of(c * N + stripe0, 8)
        pltpu.sync_copy(S.at[pl.ds(stripe0, STRIPE)], out_hbm.at[pl.ds(ob, STRIPE)])

        @pl.when(s == 0)
        def _():
            tb = pl.multiple_of(c * N + NS * STRIPE, 8)
            pltpu.sync_copy(S.at[pl.ds(NS * STRIPE, TAIL)],
                            out_hbm.at[pl.ds(tb, TAIL)])

    return functools.partial(
        pl.kernel,
        out_type=jax.ShapeDtypeStruct((2 * N, d2), jnp.float32),
        mesh=_mesh,
        scratch_types=[
            pltpu.VMEM((CH,), jnp.int32),
            pltpu.VMEM((CH,), jnp.int32),
            pltpu.VMEM((CH, d2), jnp.float32),
            pltpu.VMEM((ZROWS, d2), jnp.float32),
            pltpu.VMEM_SHARED((N, d2), jnp.float32),
            pltpu.SemaphoreType.DMA,
        ],
    )(body)


_agg_l1 = _make_agg(D_IN, True)
_agg128 = _make_agg(D_H // 2, False)


def _pool_body(h3_hbm, st_hbm, cnt_hbm, rcp_hbm, out_hbm,
               rbuf, sacc, macc, obuf, sv, cv, rv):
    c = lax.axis_index("c")
    s = lax.axis_index("s")
    wid = s * NC + c
    pltpu.sync_copy(st_hbm.at[pl.ds(wid * 8, L)], sv)
    pltpu.sync_copy(cnt_hbm.at[pl.ds(wid * 8, L)], cv)
    pltpu.sync_copy(rcp_hbm.at[pl.ds(wid * 8, L)], rv)
    lanes = lax.broadcasted_iota(jnp.int32, (L,), 0)
    zv = jnp.zeros((L,), jnp.float32)
    nv = jnp.full((L,), NEG, jnp.float32)

    def graph(gi, _):
        start = jnp.sum(jnp.where(lanes == gi, sv[...], 0))
        cnt = jnp.sum(jnp.where(lanes == gi, cv[...], 0))
        for k in range(D_OUT // L):
            sacc[pl.ds(k * L, L)] = zv
            macc[pl.ds(k * L, L)] = nv
        nch = lax.div(cnt + CHP - 1, CHP)

        def chunk(k, _):
            off = jnp.minimum(start + k * CHP, N - CHW)
            off8 = pl.multiple_of((off // 8) * 8, 8)
            pltpu.sync_copy(h3_hbm.at[pl.ds(off8, CHW)], rbuf)
            lo = start + k * CHP
            hi = jnp.minimum(lo + CHP, start + cnt)

            def row(j, _):
                r = off8 + j
                valid = (r >= lo) & (r < hi)
                for k2 in range(D_OUT // L):
                    v = rbuf[j, pl.ds(k2 * L, L)]
                    sacc[pl.ds(k2 * L, L)] = (
                        sacc[pl.ds(k2 * L, L)] + jnp.where(valid, v, 0.0))
                    macc[pl.ds(k2 * L, L)] = jnp.maximum(
                        macc[pl.ds(k2 * L, L)], jnp.where(valid, v, NEG))
                return 0
            lax.fori_loop(0, CHW, row, 0)
            return 0
        lax.fori_loop(0, nch, chunk, 0)

        rc = jnp.sum(jnp.where(lanes == gi, rv[...], 0.0))
        has = cnt > 0
        for k2 in range(D_OUT // L):
            v = sacc[pl.ds(k2 * L, L)] * rc + macc[pl.ds(k2 * L, L)]
            obuf[gi, pl.ds(k2 * L, L)] = jnp.where(has, v, 0.0)
        return 0
    lax.fori_loop(0, G // NW, graph, 0)
    pltpu.sync_copy(obuf, out_hbm.at[pl.ds(pl.multiple_of(wid * 8, 8), G // NW)])


_pool_call = functools.partial(
    pl.kernel,
    out_type=jax.ShapeDtypeStruct((G, D_OUT), jnp.float32),
    mesh=_mesh,
    compiler_params=_sc_params,
    scratch_types=[
        pltpu.VMEM((CHW, D_OUT), jnp.float32),
        pltpu.VMEM((D_OUT,), jnp.float32),
        pltpu.VMEM((D_OUT,), jnp.float32),
        pltpu.VMEM((G // NW, D_OUT), jnp.float32),
        pltpu.VMEM((L,), jnp.int32),
        pltpu.VMEM((L,), jnp.int32),
        pltpu.VMEM((L,), jnp.float32),
    ],
)(_pool_body)


R = 2000
NBLK = N // R


def _k0_body(parts_ref, x_ref, dinv_ref, xs_ref):
    deg = jnp.sum(parts_ref[...], axis=0) + 1.0
    dinv = lax.rsqrt(deg)[:, None]
    dinv_ref[...] = dinv
    xs_ref[...] = x_ref[...] * dinv


def _k0(parts, x):
    return pl.pallas_call(
        _k0_body,
        out_shape=[
            jax.ShapeDtypeStruct((N, 1), jnp.float32),
            jax.ShapeDtypeStruct((N, D_IN), jnp.float32),
        ],
    )(parts, x)


def _make_k1(d2, dout, with_stats):
    def body(*refs):
        if with_stats:
            (accA, accB, hsA, hsB, dinv_ref, w_ref, b_ref, y_ref, st_ref) = refs
        else:
            (accA, accB, hsA, hsB, dinv_ref, w_ref, b_ref, y_ref) = refs
        dinv = dinv_ref[...]
        zA = (accA[...] + hsA[...]) * dinv
        zB = (accB[...] + hsB[...]) * dinv
        w = w_ref[...]
        y = (jnp.dot(zA, w[:d2], preferred_element_type=jnp.float32)
             + jnp.dot(zB, w[d2:], preferred_element_type=jnp.float32)
             + b_ref[...])
        y_ref[...] = y
        if with_stats:
            i = pl.program_id(0)

            @pl.when(i == 0)
            def _():
                st_ref[...] = jnp.zeros_like(st_ref)

            st_ref[0:1] = st_ref[0:1] + jnp.sum(y, axis=0, keepdims=True)
            st_ref[1:2] = st_ref[1:2] + jnp.sum(y * y, axis=0, keepdims=True)

    out_specs = [pl.BlockSpec((R, dout), lambda i: (i, 0))]
    out_shape = [jax.ShapeDtypeStruct((N, dout), jnp.float32)]
    if with_stats:
        out_specs.append(pl.BlockSpec((8, dout), lambda i: (0, 0)))
        out_shape.append(jax.ShapeDtypeStruct((8, dout), jnp.float32))

    def call(acc, hs, dinv, w, b):
        return pl.pallas_call(
            body,
            grid=(NBLK,),
            in_specs=[
                pl.BlockSpec((R, d2), lambda i: (i, 0)),
                pl.BlockSpec((R, d2), lambda i: (i + NBLK, 0)),
                pl.BlockSpec((R, d2), lambda i: (i, 0)),
                pl.BlockSpec((R, d2), lambda i: (i + NBLK, 0)),
                pl.BlockSpec((R, 1), lambda i: (i, 0)),
                pl.BlockSpec((2 * d2, dout), lambda i: (0, 0)),
                pl.BlockSpec((1, dout), lambda i: (0, 0)),
            ],
            out_specs=out_specs,
            out_shape=out_shape,
        )(acc, acc, hs, hs, dinv, w, b.reshape(1, dout))
    return call


def _k1_l1_body(accA, accB, hs_ref, dinv_ref, w_ref, b_ref, y_ref, st_ref):
    z = (accA[...] + accB[...] + hs_ref[...]) * dinv_ref[...]
    y = jnp.dot(z, w_ref[...], preferred_element_type=jnp.float32) + b_ref[...]
    y_ref[...] = y
    i = pl.program_id(0)

    @pl.when(i == 0)
    def _():
        st_ref[...] = jnp.zeros_like(st_ref)

    st_ref[0:1] = st_ref[0:1] + jnp.sum(y, axis=0, keepdims=True)
    st_ref[1:2] = st_ref[1:2] + jnp.sum(y * y, axis=0, keepdims=True)


def _k1_l1(acc, hs, dinv, w, b):
    return pl.pallas_call(
        _k1_l1_body,
        grid=(NBLK,),
        in_specs=[
            pl.BlockSpec((R, D_IN), lambda i: (i, 0)),
            pl.BlockSpec((R, D_IN), lambda i: (i + NBLK, 0)),
            pl.BlockSpec((R, D_IN), lambda i: (i, 0)),
            pl.BlockSpec((R, 1), lambda i: (i, 0)),
            pl.BlockSpec((D_IN, D_H), lambda i: (0, 0)),
            pl.BlockSpec((1, D_H), lambda i: (0, 0)),
        ],
        out_specs=[
            pl.BlockSpec((R, D_H), lambda i: (i, 0)),
            pl.BlockSpec((8, D_H), lambda i: (0, 0)),
        ],
        out_shape=[
            jax.ShapeDtypeStruct((N, D_H), jnp.float32),
            jax.ShapeDtypeStruct((8, D_H), jnp.float32),
        ],
    )(acc, acc, hs, dinv, w, b.reshape(1, D_H))


_k1_l2 = _make_k1(128, D_H, True)
_k1_l3 = _make_k1(128, D_OUT, False)


def _make_k2(dout):
    half = dout // 2

    def body(y_ref, st_ref, g_ref, be_ref, dinv_ref, o_ref):
        st = st_ref[...]
        mean = st[0:1] / N
        var = st[1:2] / N - mean * mean
        inv = lax.rsqrt(var + 1e-5)
        h = g_ref[...] * (y_ref[...] - mean) * inv + be_ref[...]
        h = jnp.maximum(h, 0.0) * dinv_ref[...]
        o_ref[0] = h[:, :half]
        o_ref[1] = h[:, half:]

    def call(y, st, g, be, dinv):
        return pl.pallas_call(
            body,
            grid=(NBLK,),
            in_specs=[
                pl.BlockSpec((R, dout), lambda i: (i, 0)),
                pl.BlockSpec((8, dout), lambda i: (0, 0)),
                pl.BlockSpec((1, dout), lambda i: (0, 0)),
                pl.BlockSpec((1, dout), lambda i: (0, 0)),
                pl.BlockSpec((R, 1), lambda i: (i, 0)),
            ],
            out_specs=pl.BlockSpec((2, R, half), lambda i: (0, i, 0)),
            out_shape=jax.ShapeDtypeStruct((2, N, half), jnp.float32),
        )(y, st, g.reshape(1, dout), be.reshape(1, dout), dinv)
    return call


_k2_l1 = _make_k2(D_H)
_k2_l2 = _make_k2(D_H)


def _k3_body(b_ref, st_ref):
    i = pl.program_id(0)
    bb = b_ref[...]
    gr = lax.broadcasted_iota(jnp.int32, (1, GPAD), 1)
    eq = (bb == gr).astype(jnp.float32)
    lt = (gr > bb).astype(jnp.float32)

    @pl.when(i == 0)
    def _():
        st_ref[...] = jnp.zeros_like(st_ref)

    st_ref[0:1] = st_ref[0:1] + jnp.sum(eq, axis=0, keepdims=True)
    st_ref[1:2] = st_ref[1:2] + jnp.sum(lt, axis=0, keepdims=True)

    @pl.when(i == NBLK - 1)
    def _():
        st_ref[2:3] = 1.0 / jnp.maximum(st_ref[0:1], 1.0)


def _k3(batch2):
    return pl.pallas_call(
        _k3_body,
        grid=(NBLK,),
        in_specs=[pl.BlockSpec((R, 1), lambda i: (i, 0))],
        out_specs=pl.BlockSpec((8, GPAD), lambda i: (0, 0)),
        out_shape=jax.ShapeDtypeStruct((8, GPAD), jnp.float32),
    )(batch2)


def kernel(x, edge_index, batch, W1, b1, g1, be1, W2, b2, g2, be2, W3, b3):
    src = edge_index[0]
    dst = edge_index[1]

    parts = _deg_call(dst)
    dinv, xs = _k0(parts, x)

    acc1 = _agg_l1(xs, src, dst)
    y1, st1 = _k1_l1(acc1, xs, dinv, W1, b1)
    hs1 = _k2_l1(y1, st1, g1, be1, dinv).reshape(2 * N, D_H // 2)

    acc2 = _agg128(hs1, src, dst)
    y2, st2 = _k1_l2(acc2, hs1, dinv, W2, b2)
    hs2 = _k2_l2(y2, st2, g2, be2, dinv).reshape(2 * N, D_H // 2)

    acc3 = _agg128(hs2, src, dst)
    (h3,) = _k1_l3(acc3, hs2, dinv, W3, b3)

    st = _k3(batch.reshape(N, 1))
    counts = st[0].astype(jnp.int32)
    starts = st[1].astype(jnp.int32)
    rcp = st[2]

    return _pool_call(h3, starts, counts, rcp)


# speedup vs baseline: 16.7288x; 1.3945x over previous
import functools

import jax
import jax.numpy as jnp
from jax import lax
from jax.experimental import pallas as pl
from jax.experimental.pallas import tpu as pltpu
from jax.experimental.pallas import tpu_sc as plsc

N = 10000
E = 320000
G = 256
D_IN = 128
D_H = 256
D_OUT = 384

NC = 2
NS = 16
L = 16
NW = NC * NS

CH = 128
STRIPE = 624
ZROWS = 52
TAIL = N - NS * STRIPE
CHP = 64
CHW = 72
GPAD = 384
NEG = -3.0e38

_mesh = plsc.VectorSubcoreMesh(core_axis_name="c", subcore_axis_name="s")
_sc_params = pltpu.CompilerParams(needs_layout_passes=False)


def _deg_body(dst_hbm, out_hbm, idx_v, hist_v):
    c = lax.axis_index("c")
    s = lax.axis_index("s")
    wid = s * NC + c
    epw = E // NW
    zv = jnp.zeros((L,), jnp.float32)

    def zero(i, _):
        hist_v[pl.ds(i * L, L)] = zv
        return 0
    lax.fori_loop(0, N // L, zero, 0)

    pltpu.sync_copy(dst_hbm.at[pl.ds(wid * epw, epw)], idx_v)
    ones = jnp.ones((L,), jnp.float32)

    def body(i, _):
        iv = idx_v[pl.ds(i * L, L)]
        plsc.addupdate_scatter(hist_v, [iv], ones)
        return 0
    lax.fori_loop(0, epw // L, body, 0)

    pltpu.sync_copy(hist_v, out_hbm.at[wid])


_deg_call = functools.partial(
    pl.kernel,
    out_type=jax.ShapeDtypeStruct((NW, N), jnp.float32),
    mesh=_mesh,
    compiler_params=_sc_params,
    scratch_types=[
        pltpu.VMEM((E // NW,), jnp.int32),
        pltpu.VMEM((N,), jnp.float32),
    ],
)(_deg_body)


def _make_agg(d2, edge_split):
    npairs_total = (E // CH) // 2 // (2 if edge_split else 1)
    PQ, PR = npairs_total // NS, npairs_total % NS

    def body(hs_hbm, src_hbm, dst_hbm, out_hbm,
             sidx2, didx2, gxA, dxA, gxB, dxB, rowsA, rowsB,
             zbuf, S, isA, isB, gsA, gsB, ssA, ssB):
        c = lax.axis_index("c")
        s = lax.axis_index("s")
        zv = jnp.zeros((L,), jnp.float32)

        def zb(r, _):
            for t in range(d2 // L):
                zbuf[r, pl.ds(t * L, L)] = zv
            return 0
        lax.fori_loop(0, ZROWS, zb, 0)
        stripe0 = pl.multiple_of(s * STRIPE, 8)
        for k in range(STRIPE // ZROWS):
            pltpu.sync_copy(zbuf, S.at[pl.ds(stripe0 + k * ZROWS, ZROWS)])

        @pl.when(s == 0)
        def _():
            pltpu.sync_copy(zbuf.at[pl.ds(0, TAIL)], S.at[pl.ds(NS * STRIPE, TAIL)])
        plsc.subcore_barrier()

        npairs = PQ + (s < PR).astype(jnp.int32)
        row0 = (PQ * s + jnp.minimum(s, PR)) * 2
        if edge_split:
            row0 = row0 + c * (E // CH // 2)
        offv = jnp.broadcast_to(c * N, (L,)).astype(jnp.int32)

        def fill(off, gx, dx):
            for k in range(CH // L):
                sl = pl.ds(off + k * L, L)
                gv = sidx2[sl]
                if not edge_split:
                    gv = gv + offv
                gx[pl.ds(k * L, L)] = gv
                dx[pl.ds(k * L, L)] = didx2[sl]

        def pair(j, _):
            base = pl.multiple_of((row0 + j * 2) * CH, 8)
            ia = pltpu.async_copy(src_hbm.at[pl.ds(base, 2 * CH)], sidx2, isA)
            ib = pltpu.async_copy(dst_hbm.at[pl.ds(base, 2 * CH)], didx2, isB)
            ia.wait()
            ib.wait()
            fill(0, gxA, dxA)
            da = pltpu.async_copy(hs_hbm.at[gxA], rowsA, gsA)
            fill(CH, gxB, dxB)
            db = pltpu.async_copy(hs_hbm.at[gxB], rowsB, gsB)
            da.wait()
            sa = pltpu.async_copy(rowsA, S.at[dxA], ssA, add=True)
            db.wait()
            sc = pltpu.async_copy(rowsB, S.at[dxB], ssB, add=True)
            sa.wait()
            sc.wait()
            return 0
        lax.fori_loop(0, npairs, pair, 0)
        plsc.subcore_barrier()

        ob = pl.multiple_of(c * N + stripe0, 8)
        pltpu.sync_copy(S.at[pl.ds(stripe0, STRIPE)], out_hbm.at[pl.ds(ob, STRIPE)])

        @pl.when(s == 0)
        def _():
            tb = pl.multiple_of(c * N + NS * STRIPE, 8)
            pltpu.sync_copy(S.at[pl.ds(NS * STRIPE, TAIL)],
                            out_hbm.at[pl.ds(tb, TAIL)])

    return functools.partial(
        pl.kernel,
        out_type=jax.ShapeDtypeStruct((2 * N, d2), jnp.float32),
        mesh=_mesh,
        scratch_types=[
            pltpu.VMEM((2 * CH,), jnp.int32),
            pltpu.VMEM((2 * CH,), jnp.int32),
            pltpu.VMEM((CH,), jnp.int32),
            pltpu.VMEM((CH,), jnp.int32),
            pltpu.VMEM((CH,), jnp.int32),
            pltpu.VMEM((CH,), jnp.int32),
            pltpu.VMEM((CH, d2), jnp.float32),
            pltpu.VMEM((CH, d2), jnp.float32),
            pltpu.VMEM((ZROWS, d2), jnp.float32),
            pltpu.VMEM_SHARED((N, d2), jnp.float32),
            pltpu.SemaphoreType.DMA,
            pltpu.SemaphoreType.DMA,
            pltpu.SemaphoreType.DMA,
            pltpu.SemaphoreType.DMA,
            pltpu.SemaphoreType.DMA,
            pltpu.SemaphoreType.DMA,
        ],
    )(body)


_agg_l1 = _make_agg(D_IN, True)
_agg128 = _make_agg(D_H // 2, False)


def _pool_body(h3_hbm, st_hbm, cnt_hbm, rcp_hbm, out_hbm,
               rbuf, sacc, macc, obuf, sv, cv, rv):
    c = lax.axis_index("c")
    s = lax.axis_index("s")
    wid = s * NC + c
    pltpu.sync_copy(st_hbm.at[pl.ds(wid * 8, L)], sv)
    pltpu.sync_copy(cnt_hbm.at[pl.ds(wid * 8, L)], cv)
    pltpu.sync_copy(rcp_hbm.at[pl.ds(wid * 8, L)], rv)
    lanes = lax.broadcasted_iota(jnp.int32, (L,), 0)
    zv = jnp.zeros((L,), jnp.float32)
    nv = jnp.full((L,), NEG, jnp.float32)

    def graph(gi, _):
        start = jnp.sum(jnp.where(lanes == gi, sv[...], 0))
        cnt = jnp.sum(jnp.where(lanes == gi, cv[...], 0))
        for k in range(D_OUT // L):
            sacc[pl.ds(k * L, L)] = zv
            macc[pl.ds(k * L, L)] = nv
        nch = lax.div(cnt + CHP - 1, CHP)

        def chunk(k, _):
            off = jnp.minimum(start + k * CHP, N - CHW)
            off8 = pl.multiple_of((off // 8) * 8, 8)
            pltpu.sync_copy(h3_hbm.at[pl.ds(off8, CHW)], rbuf)
            lo = start + k * CHP
            hi = jnp.minimum(lo + CHP, start + cnt)

            def row(j, _):
                r = off8 + j
                valid = (r >= lo) & (r < hi)
                for k2 in range(D_OUT // L):
                    v = rbuf[j, pl.ds(k2 * L, L)]
                    sacc[pl.ds(k2 * L, L)] = (
                        sacc[pl.ds(k2 * L, L)] + jnp.where(valid, v, 0.0))
                    macc[pl.ds(k2 * L, L)] = jnp.maximum(
                        macc[pl.ds(k2 * L, L)], jnp.where(valid, v, NEG))
                return 0
            lax.fori_loop(0, CHW, row, 0)
            return 0
        lax.fori_loop(0, nch, chunk, 0)

        rc = jnp.sum(jnp.where(lanes == gi, rv[...], 0.0))
        has = cnt > 0
        for k2 in range(D_OUT // L):
            v = sacc[pl.ds(k2 * L, L)] * rc + macc[pl.ds(k2 * L, L)]
            obuf[gi, pl.ds(k2 * L, L)] = jnp.where(has, v, 0.0)
        return 0
    lax.fori_loop(0, G // NW, graph, 0)
    pltpu.sync_copy(obuf, out_hbm.at[pl.ds(pl.multiple_of(wid * 8, 8), G // NW)])


_pool_call = functools.partial(
    pl.kernel,
    out_type=jax.ShapeDtypeStruct((G, D_OUT), jnp.float32),
    mesh=_mesh,
    compiler_params=_sc_params,
    scratch_types=[
        pltpu.VMEM((CHW, D_OUT), jnp.float32),
        pltpu.VMEM((D_OUT,), jnp.float32),
        pltpu.VMEM((D_OUT,), jnp.float32),
        pltpu.VMEM((G // NW, D_OUT), jnp.float32),
        pltpu.VMEM((L,), jnp.int32),
        pltpu.VMEM((L,), jnp.int32),
        pltpu.VMEM((L,), jnp.float32),
    ],
)(_pool_body)


R = 2000
NBLK = N // R


def _k0_body(parts_ref, x_ref, dinv_ref, xs_ref):
    deg = jnp.sum(parts_ref[...], axis=0) + 1.0
    dinv = lax.rsqrt(deg)[:, None]
    dinv_ref[...] = dinv
    xs_ref[...] = x_ref[...] * dinv


def _k0(parts, x):
    return pl.pallas_call(
        _k0_body,
        out_shape=[
            jax.ShapeDtypeStruct((N, 1), jnp.float32),
            jax.ShapeDtypeStruct((N, D_IN), jnp.float32),
        ],
    )(parts, x)


def _make_k1(d2, dout, with_stats):
    def body(*refs):
        if with_stats:
            (accA, accB, hsA, hsB, dinv_ref, w_ref, b_ref, y_ref, st_ref) = refs
        else:
            (accA, accB, hsA, hsB, dinv_ref, w_ref, b_ref, y_ref) = refs
        dinv = dinv_ref[...]
        zA = (accA[...] + hsA[...]) * dinv
        zB = (accB[...] + hsB[...]) * dinv
        w = w_ref[...]
        y = (jnp.dot(zA, w[:d2], preferred_element_type=jnp.float32)
             + jnp.dot(zB, w[d2:], preferred_element_type=jnp.float32)
             + b_ref[...])
        y_ref[...] = y
        if with_stats:
            i = pl.program_id(0)

            @pl.when(i == 0)
            def _():
                st_ref[...] = jnp.zeros_like(st_ref)

            st_ref[0:1] = st_ref[0:1] + jnp.sum(y, axis=0, keepdims=True)
            st_ref[1:2] = st_ref[1:2] + jnp.sum(y * y, axis=0, keepdims=True)

    out_specs = [pl.BlockSpec((R, dout), lambda i: (i, 0))]
    out_shape = [jax.ShapeDtypeStruct((N, dout), jnp.float32)]
    if with_stats:
        out_specs.append(pl.BlockSpec((8, dout), lambda i: (0, 0)))
        out_shape.append(jax.ShapeDtypeStruct((8, dout), jnp.float32))

    def call(acc, hs, dinv, w, b):
        return pl.pallas_call(
            body,
            grid=(NBLK,),
            in_specs=[
                pl.BlockSpec((R, d2), lambda i: (i, 0)),
                pl.BlockSpec((R, d2), lambda i: (i + NBLK, 0)),
                pl.BlockSpec((R, d2), lambda i: (i, 0)),
                pl.BlockSpec((R, d2), lambda i: (i + NBLK, 0)),
                pl.BlockSpec((R, 1), lambda i: (i, 0)),
                pl.BlockSpec((2 * d2, dout), lambda i: (0, 0)),
                pl.BlockSpec((1, dout), lambda i: (0, 0)),
            ],
            out_specs=out_specs,
            out_shape=out_shape,
        )(acc, acc, hs, hs, dinv, w, b.reshape(1, dout))
    return call


def _k1_l1_body(accA, accB, hs_ref, dinv_ref, w_ref, b_ref, y_ref, st_ref):
    z = (accA[...] + accB[...] + hs_ref[...]) * dinv_ref[...]
    y = jnp.dot(z, w_ref[...], preferred_element_type=jnp.float32) + b_ref[...]
    y_ref[...] = y
    i = pl.program_id(0)

    @pl.when(i == 0)
    def _():
        st_ref[...] = jnp.zeros_like(st_ref)

    st_ref[0:1] = st_ref[0:1] + jnp.sum(y, axis=0, keepdims=True)
    st_ref[1:2] = st_ref[1:2] + jnp.sum(y * y, axis=0, keepdims=True)


def _k1_l1(acc, hs, dinv, w, b):
    return pl.pallas_call(
        _k1_l1_body,
        grid=(NBLK,),
        in_specs=[
            pl.BlockSpec((R, D_IN), lambda i: (i, 0)),
            pl.BlockSpec((R, D_IN), lambda i: (i + NBLK, 0)),
            pl.BlockSpec((R, D_IN), lambda i: (i, 0)),
            pl.BlockSpec((R, 1), lambda i: (i, 0)),
            pl.BlockSpec((D_IN, D_H), lambda i: (0, 0)),
            pl.BlockSpec((1, D_H), lambda i: (0, 0)),
        ],
        out_specs=[
            pl.BlockSpec((R, D_H), lambda i: (i, 0)),
            pl.BlockSpec((8, D_H), lambda i: (0, 0)),
        ],
        out_shape=[
            jax.ShapeDtypeStruct((N, D_H), jnp.float32),
            jax.ShapeDtypeStruct((8, D_H), jnp.float32),
        ],
    )(acc, acc, hs, dinv, w, b.reshape(1, D_H))


_k1_l2 = _make_k1(128, D_H, True)
_k1_l3 = _make_k1(128, D_OUT, False)


def _make_k2(dout):
    half = dout // 2

    def body(y_ref, st_ref, g_ref, be_ref, dinv_ref, o_ref):
        st = st_ref[...]
        mean = st[0:1] / N
        var = st[1:2] / N - mean * mean
        inv = lax.rsqrt(var + 1e-5)
        h = g_ref[...] * (y_ref[...] - mean) * inv + be_ref[...]
        h = jnp.maximum(h, 0.0) * dinv_ref[...]
        o_ref[0] = h[:, :half]
        o_ref[1] = h[:, half:]

    def call(y, st, g, be, dinv):
        return pl.pallas_call(
            body,
            grid=(NBLK,),
            in_specs=[
                pl.BlockSpec((R, dout), lambda i: (i, 0)),
                pl.BlockSpec((8, dout), lambda i: (0, 0)),
                pl.BlockSpec((1, dout), lambda i: (0, 0)),
                pl.BlockSpec((1, dout), lambda i: (0, 0)),
                pl.BlockSpec((R, 1), lambda i: (i, 0)),
            ],
            out_specs=pl.BlockSpec((2, R, half), lambda i: (0, i, 0)),
            out_shape=jax.ShapeDtypeStruct((2, N, half), jnp.float32),
        )(y, st, g.reshape(1, dout), be.reshape(1, dout), dinv)
    return call


_k2_l1 = _make_k2(D_H)
_k2_l2 = _make_k2(D_H)


def _k3_body(b_ref, st_ref):
    i = pl.program_id(0)
    bb = b_ref[...]
    gr = lax.broadcasted_iota(jnp.int32, (1, GPAD), 1)
    eq = (bb == gr).astype(jnp.float32)
    lt = (gr > bb).astype(jnp.float32)

    @pl.when(i == 0)
    def _():
        st_ref[...] = jnp.zeros_like(st_ref)

    st_ref[0:1] = st_ref[0:1] + jnp.sum(eq, axis=0, keepdims=True)
    st_ref[1:2] = st_ref[1:2] + jnp.sum(lt, axis=0, keepdims=True)

    @pl.when(i == NBLK - 1)
    def _():
        st_ref[2:3] = 1.0 / jnp.maximum(st_ref[0:1], 1.0)


def _k3(batch2):
    return pl.pallas_call(
        _k3_body,
        grid=(NBLK,),
        in_specs=[pl.BlockSpec((R, 1), lambda i: (i, 0))],
        out_specs=pl.BlockSpec((8, GPAD), lambda i: (0, 0)),
        out_shape=jax.ShapeDtypeStruct((8, GPAD), jnp.float32),
    )(batch2)


def kernel(x, edge_index, batch, W1, b1, g1, be1, W2, b2, g2, be2, W3, b3):
    src = edge_index[0]
    dst = edge_index[1]

    parts = _deg_call(dst)
    dinv, xs = _k0(parts, x)

    acc1 = _agg_l1(xs, src, dst)
    y1, st1 = _k1_l1(acc1, xs, dinv, W1, b1)
    hs1 = _k2_l1(y1, st1, g1, be1, dinv).reshape(2 * N, D_H // 2)

    acc2 = _agg128(hs1, src, dst)
    y2, st2 = _k1_l2(acc2, hs1, dinv, W2, b2)
    hs2 = _k2_l2(y2, st2, g2, be2, dinv).reshape(2 * N, D_H // 2)

    acc3 = _agg128(hs2, src, dst)
    (h3,) = _k1_l3(acc3, hs2, dinv, W3, b3)

    st = _k3(batch.reshape(N, 1))
    counts = st[0].astype(jnp.int32)
    starts = st[1].astype(jnp.int32)
    rcp = st[2]

    return _pool_call(h3, starts, counts, rcp)


# speedup vs baseline: 18.9002x; 1.1298x over previous
import functools

import jax
import jax.numpy as jnp
from jax import lax
from jax.experimental import pallas as pl
from jax.experimental.pallas import tpu as pltpu
from jax.experimental.pallas import tpu_sc as plsc

N = 10000
E = 320000
G = 256
D_IN = 128
D_H = 256
D_OUT = 384

NC = 2
NS = 16
L = 16
NW = NC * NS

CH = 128
STRIPE = 624
ZROWS = 52
TAIL = N - NS * STRIPE
CHP = 64
CHW = 72
GP = 8
GPAD = 384
NEG = -3.0e38

_mesh = plsc.VectorSubcoreMesh(core_axis_name="c", subcore_axis_name="s")
_sc_params = pltpu.CompilerParams(needs_layout_passes=False)


def _deg_body(dst_hbm, out_hbm, idx_v, hist_v):
    c = lax.axis_index("c")
    s = lax.axis_index("s")
    wid = s * NC + c
    epw = E // NW
    zv = jnp.zeros((L,), jnp.float32)

    def zero(i, _):
        hist_v[pl.ds(i * L, L)] = zv
        return 0
    lax.fori_loop(0, N // L, zero, 0)

    pltpu.sync_copy(dst_hbm.at[pl.ds(wid * epw, epw)], idx_v)
    ones = jnp.ones((L,), jnp.float32)

    def body(i, _):
        iv = idx_v[pl.ds(i * L, L)]
        plsc.addupdate_scatter(hist_v, [iv], ones)
        return 0
    lax.fori_loop(0, epw // L, body, 0)

    pltpu.sync_copy(hist_v, out_hbm.at[wid])


_deg_call = functools.partial(
    pl.kernel,
    out_type=jax.ShapeDtypeStruct((NW, N), jnp.float32),
    mesh=_mesh,
    compiler_params=_sc_params,
    scratch_types=[
        pltpu.VMEM((E // NW,), jnp.int32),
        pltpu.VMEM((N,), jnp.float32),
    ],
)(_deg_body)


def _make_agg(d2, edge_split):
    npairs_total = (E // CH) // 2 // (2 if edge_split else 1)
    PQ, PR = npairs_total // NS, npairs_total % NS

    def body(hs_hbm, src_hbm, dst_hbm, out_hbm,
             sidx2, didx2, gxA, dxA, gxB, dxB, rowsA, rowsB,
             zbuf, S, isA, isB, gsA, gsB, ssA, ssB):
        c = lax.axis_index("c")
        s = lax.axis_index("s")
        zv = jnp.zeros((L,), jnp.float32)

        def zb(r, _):
            for t in range(d2 // L):
                zbuf[r, pl.ds(t * L, L)] = zv
            return 0
        lax.fori_loop(0, ZROWS, zb, 0)
        stripe0 = pl.multiple_of(s * STRIPE, 8)
        for k in range(STRIPE // ZROWS):
            pltpu.sync_copy(zbuf, S.at[pl.ds(stripe0 + k * ZROWS, ZROWS)])

        @pl.when(s == 0)
        def _():
            pltpu.sync_copy(zbuf.at[pl.ds(0, TAIL)], S.at[pl.ds(NS * STRIPE, TAIL)])
        plsc.subcore_barrier()

        npairs = PQ + (s < PR).astype(jnp.int32)
        row0 = (PQ * s + jnp.minimum(s, PR)) * 2
        if edge_split:
            row0 = row0 + c * (E // CH // 2)
        offv = jnp.broadcast_to(c * N, (L,)).astype(jnp.int32)

        def fill(off, gx, dx):
            for k in range(CH // L):
                sl = pl.ds(off + k * L, L)
                gv = sidx2[sl]
                if not edge_split:
                    gv = gv + offv
                gx[pl.ds(k * L, L)] = gv
                dx[pl.ds(k * L, L)] = didx2[sl]

        GE = GP * 2 * CH

        def group(g, _):
            base = (row0 + g * GP * 2) * CH
            gb = pl.multiple_of(jnp.minimum(base, E - GE), 8)
            delta = base - gb
            ia = pltpu.async_copy(src_hbm.at[pl.ds(gb, GE)], sidx2, isA)
            ib = pltpu.async_copy(dst_hbm.at[pl.ds(gb, GE)], didx2, isB)
            ia.wait()
            ib.wait()
            npr = jnp.minimum(GP, npairs - g * GP)

            def pair(p, _):
                off0 = delta + p * 2 * CH
                fill(off0, gxA, dxA)
                da = pltpu.async_copy(hs_hbm.at[gxA], rowsA, gsA)
                fill(off0 + CH, gxB, dxB)
                db = pltpu.async_copy(hs_hbm.at[gxB], rowsB, gsB)
                da.wait()
                sa = pltpu.async_copy(rowsA, S.at[dxA], ssA, add=True)
                db.wait()
                sc = pltpu.async_copy(rowsB, S.at[dxB], ssB, add=True)
                sa.wait()
                sc.wait()
                return 0
            lax.fori_loop(0, npr, pair, 0)
            return 0
        lax.fori_loop(0, lax.div(npairs + GP - 1, GP), group, 0)
        plsc.subcore_barrier()

        ob = pl.multiple_of(c * N + stripe0, 8)
        pltpu.sync_copy(S.at[pl.ds(stripe0, STRIPE)], out_hbm.at[pl.ds(ob, STRIPE)])

        @pl.when(s == 0)
        def _():
            tb = pl.multiple_of(c * N + NS * STRIPE, 8)
            pltpu.sync_copy(S.at[pl.ds(NS * STRIPE, TAIL)],
                            out_hbm.at[pl.ds(tb, TAIL)])

    return functools.partial(
        pl.kernel,
        out_type=jax.ShapeDtypeStruct((2 * N, d2), jnp.float32),
        mesh=_mesh,
        scratch_types=[
            pltpu.VMEM((GP * 2 * CH,), jnp.int32),
            pltpu.VMEM((GP * 2 * CH,), jnp.int32),
            pltpu.VMEM((CH,), jnp.int32),
            pltpu.VMEM((CH,), jnp.int32),
            pltpu.VMEM((CH,), jnp.int32),
            pltpu.VMEM((CH,), jnp.int32),
            pltpu.VMEM((CH, d2), jnp.float32),
            pltpu.VMEM((CH, d2), jnp.float32),
            pltpu.VMEM((ZROWS, d2), jnp.float32),
            pltpu.VMEM_SHARED((N, d2), jnp.float32),
            pltpu.SemaphoreType.DMA,
            pltpu.SemaphoreType.DMA,
            pltpu.SemaphoreType.DMA,
            pltpu.SemaphoreType.DMA,
            pltpu.SemaphoreType.DMA,
            pltpu.SemaphoreType.DMA,
        ],
    )(body)


_agg_l1 = _make_agg(D_IN, True)
_agg128 = _make_agg(D_H // 2, False)


def _pool_body(h3_hbm, st_hbm, cnt_hbm, rcp_hbm, out_hbm,
               rbuf, sacc, macc, obuf, sv, cv, rv):
    c = lax.axis_index("c")
    s = lax.axis_index("s")
    wid = s * NC + c
    pltpu.sync_copy(st_hbm.at[pl.ds(wid * 8, L)], sv)
    pltpu.sync_copy(cnt_hbm.at[pl.ds(wid * 8, L)], cv)
    pltpu.sync_copy(rcp_hbm.at[pl.ds(wid * 8, L)], rv)
    lanes = lax.broadcasted_iota(jnp.int32, (L,), 0)
    zv = jnp.zeros((L,), jnp.float32)
    nv = jnp.full((L,), NEG, jnp.float32)

    def graph(gi, _):
        start = jnp.sum(jnp.where(lanes == gi, sv[...], 0))
        cnt = jnp.sum(jnp.where(lanes == gi, cv[...], 0))
        for k in range(D_OUT // L):
            sacc[pl.ds(k * L, L)] = zv
            macc[pl.ds(k * L, L)] = nv
        nch = lax.div(cnt + CHP - 1, CHP)

        def chunk(k, _):
            off = jnp.minimum(start + k * CHP, N - CHW)
            off8 = pl.multiple_of((off // 8) * 8, 8)
            pltpu.sync_copy(h3_hbm.at[pl.ds(off8, CHW)], rbuf)
            lo = start + k * CHP
            hi = jnp.minimum(lo + CHP, start + cnt)
            j0 = lo - off8
            j1 = hi - off8
            for k2 in range(D_OUT // L):
                sl = pl.ds(k2 * L, L)

                def rloop(j, carry):
                    sv, mv = carry
                    v = rbuf[j, sl]
                    return (sv + v, jnp.maximum(mv, v))
                sv, mv = lax.fori_loop(j0, j1, rloop, (zv, nv))
                sacc[sl] = sacc[sl] + sv
                macc[sl] = jnp.maximum(macc[sl], mv)
            return 0
        lax.fori_loop(0, nch, chunk, 0)

        rc = jnp.sum(jnp.where(lanes == gi, rv[...], 0.0))
        has = cnt > 0
        for k2 in range(D_OUT // L):
            v = sacc[pl.ds(k2 * L, L)] * rc + macc[pl.ds(k2 * L, L)]
            obuf[gi, pl.ds(k2 * L, L)] = jnp.where(has, v, 0.0)
        return 0
    lax.fori_loop(0, G // NW, graph, 0)
    pltpu.sync_copy(obuf, out_hbm.at[pl.ds(pl.multiple_of(wid * 8, 8), G // NW)])


_pool_call = functools.partial(
    pl.kernel,
    out_type=jax.ShapeDtypeStruct((G, D_OUT), jnp.float32),
    mesh=_mesh,
    compiler_params=_sc_params,
    scratch_types=[
        pltpu.VMEM((CHW, D_OUT), jnp.float32),
        pltpu.VMEM((D_OUT,), jnp.float32),
        pltpu.VMEM((D_OUT,), jnp.float32),
        pltpu.VMEM((G // NW, D_OUT), jnp.float32),
        pltpu.VMEM((L,), jnp.int32),
        pltpu.VMEM((L,), jnp.int32),
        pltpu.VMEM((L,), jnp.float32),
    ],
)(_pool_body)


R = 2000
NBLK = N // R


def _k0_body(parts_ref, x_ref, dinv_ref, xs_ref):
    deg = jnp.sum(parts_ref[...], axis=0) + 1.0
    dinv = lax.rsqrt(deg)[:, None]
    dinv_ref[...] = dinv
    xs_ref[...] = x_ref[...] * dinv


def _k0(parts, x):
    return pl.pallas_call(
        _k0_body,
        out_shape=[
            jax.ShapeDtypeStruct((N, 1), jnp.float32),
            jax.ShapeDtypeStruct((N, D_IN), jnp.float32),
        ],
    )(parts, x)


def _make_k1(d2, dout, with_stats):
    def body(*refs):
        if with_stats:
            (accA, accB, hsA, hsB, dinv_ref, w_ref, b_ref, y_ref, st_ref) = refs
        else:
            (accA, accB, hsA, hsB, dinv_ref, w_ref, b_ref, y_ref) = refs
        dinv = dinv_ref[...]
        zA = (accA[...] + hsA[...]) * dinv
        zB = (accB[...] + hsB[...]) * dinv
        w = w_ref[...]
        y = (jnp.dot(zA, w[:d2], preferred_element_type=jnp.float32)
             + jnp.dot(zB, w[d2:], preferred_element_type=jnp.float32)
             + b_ref[...])
        y_ref[...] = y
        if with_stats:
            i = pl.program_id(0)

            @pl.when(i == 0)
            def _():
                st_ref[...] = jnp.zeros_like(st_ref)

            st_ref[0:1] = st_ref[0:1] + jnp.sum(y, axis=0, keepdims=True)
            st_ref[1:2] = st_ref[1:2] + jnp.sum(y * y, axis=0, keepdims=True)

    out_specs = [pl.BlockSpec((R, dout), lambda i: (i, 0))]
    out_shape = [jax.ShapeDtypeStruct((N, dout), jnp.float32)]
    if with_stats:
        out_specs.append(pl.BlockSpec((8, dout), lambda i: (0, 0)))
        out_shape.append(jax.ShapeDtypeStruct((8, dout), jnp.float32))

    def call(acc, hs, dinv, w, b):
        return pl.pallas_call(
            body,
            grid=(NBLK,),
            in_specs=[
                pl.BlockSpec((R, d2), lambda i: (i, 0)),
                pl.BlockSpec((R, d2), lambda i: (i + NBLK, 0)),
                pl.BlockSpec((R, d2), lambda i: (i, 0)),
                pl.BlockSpec((R, d2), lambda i: (i + NBLK, 0)),
                pl.BlockSpec((R, 1), lambda i: (i, 0)),
                pl.BlockSpec((2 * d2, dout), lambda i: (0, 0)),
                pl.BlockSpec((1, dout), lambda i: (0, 0)),
            ],
            out_specs=out_specs,
            out_shape=out_shape,
        )(acc, acc, hs, hs, dinv, w, b.reshape(1, dout))
    return call


def _k1_l1_body(accA, accB, hs_ref, dinv_ref, w_ref, b_ref, y_ref, st_ref):
    z = (accA[...] + accB[...] + hs_ref[...]) * dinv_ref[...]
    y = jnp.dot(z, w_ref[...], preferred_element_type=jnp.float32) + b_ref[...]
    y_ref[...] = y
    i = pl.program_id(0)

    @pl.when(i == 0)
    def _():
        st_ref[...] = jnp.zeros_like(st_ref)

    st_ref[0:1] = st_ref[0:1] + jnp.sum(y, axis=0, keepdims=True)
    st_ref[1:2] = st_ref[1:2] + jnp.sum(y * y, axis=0, keepdims=True)


def _k1_l1(acc, hs, dinv, w, b):
    return pl.pallas_call(
        _k1_l1_body,
        grid=(NBLK,),
        in_specs=[
            pl.BlockSpec((R, D_IN), lambda i: (i, 0)),
            pl.BlockSpec((R, D_IN), lambda i: (i + NBLK, 0)),
            pl.BlockSpec((R, D_IN), lambda i: (i, 0)),
            pl.BlockSpec((R, 1), lambda i: (i, 0)),
            pl.BlockSpec((D_IN, D_H), lambda i: (0, 0)),
            pl.BlockSpec((1, D_H), lambda i: (0, 0)),
        ],
        out_specs=[
            pl.BlockSpec((R, D_H), lambda i: (i, 0)),
            pl.BlockSpec((8, D_H), lambda i: (0, 0)),
        ],
        out_shape=[
            jax.ShapeDtypeStruct((N, D_H), jnp.float32),
            jax.ShapeDtypeStruct((8, D_H), jnp.float32),
        ],
    )(acc, acc, hs, dinv, w, b.reshape(1, D_H))


_k1_l2 = _make_k1(128, D_H, True)
_k1_l3 = _make_k1(128, D_OUT, False)


def _make_k2(dout):
    half = dout // 2

    def body(y_ref, st_ref, g_ref, be_ref, dinv_ref, o_ref):
        st = st_ref[...]
        mean = st[0:1] / N
        var = st[1:2] / N - mean * mean
        inv = lax.rsqrt(var + 1e-5)
        h = g_ref[...] * (y_ref[...] - mean) * inv + be_ref[...]
        h = jnp.maximum(h, 0.0) * dinv_ref[...]
        o_ref[0] = h[:, :half]
        o_ref[1] = h[:, half:]

    def call(y, st, g, be, dinv):
        return pl.pallas_call(
            body,
            grid=(NBLK,),
            in_specs=[
                pl.BlockSpec((R, dout), lambda i: (i, 0)),
                pl.BlockSpec((8, dout), lambda i: (0, 0)),
                pl.BlockSpec((1, dout), lambda i: (0, 0)),
                pl.BlockSpec((1, dout), lambda i: (0, 0)),
                pl.BlockSpec((R, 1), lambda i: (i, 0)),
            ],
            out_specs=pl.BlockSpec((2, R, half), lambda i: (0, i, 0)),
            out_shape=jax.ShapeDtypeStruct((2, N, half), jnp.float32),
        )(y, st, g.reshape(1, dout), be.reshape(1, dout), dinv)
    return call


_k2_l1 = _make_k2(D_H)
_k2_l2 = _make_k2(D_H)


def _k3_body(b_ref, st_ref):
    i = pl.program_id(0)
    bb = b_ref[...]
    gr = lax.broadcasted_iota(jnp.int32, (1, GPAD), 1)
    eq = (bb == gr).astype(jnp.float32)
    lt = (gr > bb).astype(jnp.float32)

    @pl.when(i == 0)
    def _():
        st_ref[...] = jnp.zeros_like(st_ref)

    st_ref[0:1] = st_ref[0:1] + jnp.sum(eq, axis=0, keepdims=True)
    st_ref[1:2] = st_ref[1:2] + jnp.sum(lt, axis=0, keepdims=True)

    @pl.when(i == NBLK - 1)
    def _():
        st_ref[2:3] = 1.0 / jnp.maximum(st_ref[0:1], 1.0)


def _k3(batch2):
    return pl.pallas_call(
        _k3_body,
        grid=(NBLK,),
        in_specs=[pl.BlockSpec((R, 1), lambda i: (i, 0))],
        out_specs=pl.BlockSpec((8, GPAD), lambda i: (0, 0)),
        out_shape=jax.ShapeDtypeStruct((8, GPAD), jnp.float32),
    )(batch2)


def kernel(x, edge_index, batch, W1, b1, g1, be1, W2, b2, g2, be2, W3, b3):
    src = edge_index[0]
    dst = edge_index[1]

    parts = _deg_call(dst)
    dinv, xs = _k0(parts, x)

    acc1 = _agg_l1(xs, src, dst)
    y1, st1 = _k1_l1(acc1, xs, dinv, W1, b1)
    hs1 = _k2_l1(y1, st1, g1, be1, dinv).reshape(2 * N, D_H // 2)

    acc2 = _agg128(hs1, src, dst)
    y2, st2 = _k1_l2(acc2, hs1, dinv, W2, b2)
    hs2 = _k2_l2(y2, st2, g2, be2, dinv).reshape(2 * N, D_H // 2)

    acc3 = _agg128(hs2, src, dst)
    (h3,) = _k1_l3(acc3, hs2, dinv, W3, b3)

    st = _k3(batch.reshape(N, 1))
    counts = st[0].astype(jnp.int32)
    starts = st[1].astype(jnp.int32)
    rcp = st[2]

    return _pool_call(h3, starts, counts, rcp)


# speedup vs baseline: 19.6756x; 1.0410x over previous
import functools

import jax
import jax.numpy as jnp
from jax import lax
from jax.experimental import pallas as pl
from jax.experimental.pallas import tpu as pltpu
from jax.experimental.pallas import tpu_sc as plsc

N = 10000
E = 320000
G = 256
D_IN = 128
D_H = 256
D_OUT = 384

NC = 2
NS = 16
L = 16
NW = NC * NS

CH = 128
STRIPE = 624
ZROWS = 52
TAIL = N - NS * STRIPE
CHP = 64
CHW = 72
GP = 8
GPAD = 384
NEG = -3.0e38

_mesh = plsc.VectorSubcoreMesh(core_axis_name="c", subcore_axis_name="s")
_sc_params = pltpu.CompilerParams(needs_layout_passes=False)


def _deg_body(dst_hbm, out_hbm, idx_v, hist_v):
    c = lax.axis_index("c")
    s = lax.axis_index("s")
    wid = s * NC + c
    epw = E // NW
    zv = jnp.zeros((L,), jnp.float32)

    def zero(i, _):
        hist_v[pl.ds(i * L, L)] = zv
        return 0
    lax.fori_loop(0, N // L, zero, 0)

    pltpu.sync_copy(dst_hbm.at[pl.ds(wid * epw, epw)], idx_v)
    ones = jnp.ones((L,), jnp.float32)

    def body(i, _):
        iv = idx_v[pl.ds(i * L, L)]
        plsc.addupdate_scatter(hist_v, [iv], ones)
        return 0
    lax.fori_loop(0, epw // L, body, 0)

    pltpu.sync_copy(hist_v, out_hbm.at[wid])


_deg_call = functools.partial(
    pl.kernel,
    out_type=jax.ShapeDtypeStruct((NW, N), jnp.float32),
    mesh=_mesh,
    compiler_params=_sc_params,
    scratch_types=[
        pltpu.VMEM((E // NW,), jnp.int32),
        pltpu.VMEM((N,), jnp.float32),
    ],
)(_deg_body)


def _make_agg(d2, edge_split):
    npairs_total = (E // CH) // 2 // (2 if edge_split else 1)
    PQ, PR = npairs_total // NS, npairs_total % NS

    def body(hs_hbm, src_hbm, dst_hbm, out_hbm,
             sidx2, didx2, gxA, dxA, gxB, dxB, rowsA, rowsB,
             zbuf, S, isA, isB, gsA, gsB, ssA, ssB):
        c = lax.axis_index("c")
        s = lax.axis_index("s")
        zv = jnp.zeros((L,), jnp.float32)

        def zb(r, _):
            for t in range(d2 // L):
                zbuf[r, pl.ds(t * L, L)] = zv
            return 0
        lax.fori_loop(0, ZROWS, zb, 0)
        stripe0 = pl.multiple_of(s * STRIPE, 8)
        for k in range(STRIPE // ZROWS):
            pltpu.sync_copy(zbuf, S.at[pl.ds(stripe0 + k * ZROWS, ZROWS)])

        @pl.when(s == 0)
        def _():
            pltpu.sync_copy(zbuf.at[pl.ds(0, TAIL)], S.at[pl.ds(NS * STRIPE, TAIL)])
        plsc.subcore_barrier()

        npairs = PQ + (s < PR).astype(jnp.int32)
        row0 = (PQ * s + jnp.minimum(s, PR)) * 2
        if edge_split:
            row0 = row0 + c * (E // CH // 2)
        offv = jnp.broadcast_to(c * N, (L,)).astype(jnp.int32)

        def fillg(off, gx):
            for k in range(CH // L):
                gv = sidx2[pl.ds(off + k * L, L)]
                if not edge_split:
                    gv = gv + offv
                gx[pl.ds(k * L, L)] = gv

        def fill_d(off, dx):
            for k in range(CH // L):
                dx[pl.ds(k * L, L)] = didx2[pl.ds(off + k * L, L)]

        GE = GP * 2 * CH

        def group(g, _):
            base = (row0 + g * GP * 2) * CH
            gb = pl.multiple_of(jnp.minimum(base, E - GE), 8)
            delta = base - gb
            ia = pltpu.async_copy(src_hbm.at[pl.ds(gb, GE)], sidx2, isA)
            ib = pltpu.async_copy(dst_hbm.at[pl.ds(gb, GE)], didx2, isB)
            ia.wait()
            ib.wait()
            npr = jnp.minimum(GP, npairs - g * GP)

            def pair(p, _):
                off0 = delta + p * 2 * CH
                jg = g * GP + p

                @pl.when(jg > 0)
                def _():
                    pltpu.make_async_copy(rowsA, S.at[dxA], ssA).wait()
                fillg(off0, gxA)
                fill_d(off0, dxA)
                da = pltpu.async_copy(hs_hbm.at[gxA], rowsA, gsA)

                @pl.when(jg > 0)
                def _():
                    pltpu.make_async_copy(rowsB, S.at[dxB], ssB).wait()
                fillg(off0 + CH, gxB)
                fill_d(off0 + CH, dxB)
                db = pltpu.async_copy(hs_hbm.at[gxB], rowsB, gsB)
                da.wait()
                pltpu.async_copy(rowsA, S.at[dxA], ssA, add=True)
                db.wait()
                pltpu.async_copy(rowsB, S.at[dxB], ssB, add=True)
                return 0
            lax.fori_loop(0, npr, pair, 0)
            return 0
        lax.fori_loop(0, lax.div(npairs + GP - 1, GP), group, 0)
        pltpu.make_async_copy(rowsA, S.at[dxA], ssA).wait()
        pltpu.make_async_copy(rowsB, S.at[dxB], ssB).wait()
        plsc.subcore_barrier()

        ob = pl.multiple_of(c * N + stripe0, 8)
        pltpu.sync_copy(S.at[pl.ds(stripe0, STRIPE)], out_hbm.at[pl.ds(ob, STRIPE)])

        @pl.when(s == 0)
        def _():
            tb = pl.multiple_of(c * N + NS * STRIPE, 8)
            pltpu.sync_copy(S.at[pl.ds(NS * STRIPE, TAIL)],
                            out_hbm.at[pl.ds(tb, TAIL)])

    return functools.partial(
        pl.kernel,
        out_type=jax.ShapeDtypeStruct((2 * N, d2), jnp.float32),
        mesh=_mesh,
        scratch_types=[
            pltpu.VMEM((GP * 2 * CH,), jnp.int32),
            pltpu.VMEM((GP * 2 * CH,), jnp.int32),
            pltpu.VMEM((CH,), jnp.int32),
            pltpu.VMEM((CH,), jnp.int32),
            pltpu.VMEM((CH,), jnp.int32),
            pltpu.VMEM((CH,), jnp.int32),
            pltpu.VMEM((CH, d2), jnp.float32),
            pltpu.VMEM((CH, d2), jnp.float32),
            pltpu.VMEM((ZROWS, d2), jnp.float32),
            pltpu.VMEM_SHARED((N, d2), jnp.float32),
            pltpu.SemaphoreType.DMA,
            pltpu.SemaphoreType.DMA,
            pltpu.SemaphoreType.DMA,
            pltpu.SemaphoreType.DMA,
            pltpu.SemaphoreType.DMA,
            pltpu.SemaphoreType.DMA,
        ],
    )(body)


_agg_l1 = _make_agg(D_IN, True)
_agg128 = _make_agg(D_H // 2, False)


def _pool_body(h3_hbm, st_hbm, cnt_hbm, rcp_hbm, out_hbm,
               rbuf, sacc, macc, obuf, sv, cv, rv):
    c = lax.axis_index("c")
    s = lax.axis_index("s")
    wid = s * NC + c
    pltpu.sync_copy(st_hbm.at[pl.ds(wid * 8, L)], sv)
    pltpu.sync_copy(cnt_hbm.at[pl.ds(wid * 8, L)], cv)
    pltpu.sync_copy(rcp_hbm.at[pl.ds(wid * 8, L)], rv)
    lanes = lax.broadcasted_iota(jnp.int32, (L,), 0)
    zv = jnp.zeros((L,), jnp.float32)
    nv = jnp.full((L,), NEG, jnp.float32)

    def graph(gi, _):
        start = jnp.sum(jnp.where(lanes == gi, sv[...], 0))
        cnt = jnp.sum(jnp.where(lanes == gi, cv[...], 0))
        for k in range(D_OUT // L):
            sacc[pl.ds(k * L, L)] = zv
            macc[pl.ds(k * L, L)] = nv
        nch = lax.div(cnt + CHP - 1, CHP)

        def chunk(k, _):
            off = jnp.minimum(start + k * CHP, N - CHW)
            off8 = pl.multiple_of((off // 8) * 8, 8)
            pltpu.sync_copy(h3_hbm.at[pl.ds(off8, CHW)], rbuf)
            lo = start + k * CHP
            hi = jnp.minimum(lo + CHP, start + cnt)
            j0 = lo - off8
            j1 = hi - off8
            for k2 in range(D_OUT // L):
                sl = pl.ds(k2 * L, L)

                def rloop(j, carry):
                    sv, mv = carry
                    v = rbuf[j, sl]
                    return (sv + v, jnp.maximum(mv, v))
                sv, mv = lax.fori_loop(j0, j1, rloop, (zv, nv))
                sacc[sl] = sacc[sl] + sv
                macc[sl] = jnp.maximum(macc[sl], mv)
            return 0
        lax.fori_loop(0, nch, chunk, 0)

        rc = jnp.sum(jnp.where(lanes == gi, rv[...], 0.0))
        has = cnt > 0
        for k2 in range(D_OUT // L):
            v = sacc[pl.ds(k2 * L, L)] * rc + macc[pl.ds(k2 * L, L)]
            obuf[gi, pl.ds(k2 * L, L)] = jnp.where(has, v, 0.0)
        return 0
    lax.fori_loop(0, G // NW, graph, 0)
    pltpu.sync_copy(obuf, out_hbm.at[pl.ds(pl.multiple_of(wid * 8, 8), G // NW)])


_pool_call = functools.partial(
    pl.kernel,
    out_type=jax.ShapeDtypeStruct((G, D_OUT), jnp.float32),
    mesh=_mesh,
    compiler_params=_sc_params,
    scratch_types=[
        pltpu.VMEM((CHW, D_OUT), jnp.float32),
        pltpu.VMEM((D_OUT,), jnp.float32),
        pltpu.VMEM((D_OUT,), jnp.float32),
        pltpu.VMEM((G // NW, D_OUT), jnp.float32),
        pltpu.VMEM((L,), jnp.int32),
        pltpu.VMEM((L,), jnp.int32),
        pltpu.VMEM((L,), jnp.float32),
    ],
)(_pool_body)


R = 2000
NBLK = N // R


def _k0_body(parts_ref, x_ref, dinv_ref, xs_ref):
    deg = jnp.sum(parts_ref[...], axis=0) + 1.0
    dinv = lax.rsqrt(deg)[:, None]
    dinv_ref[...] = dinv
    xs_ref[...] = x_ref[...] * dinv


def _k0(parts, x):
    return pl.pallas_call(
        _k0_body,
        out_shape=[
            jax.ShapeDtypeStruct((N, 1), jnp.float32),
            jax.ShapeDtypeStruct((N, D_IN), jnp.float32),
        ],
    )(parts, x)


def _make_k1(d2, dout, with_stats):
    def body(*refs):
        if with_stats:
            (accA, accB, hsA, hsB, dinv_ref, w_ref, b_ref, y_ref, st_ref) = refs
        else:
            (accA, accB, hsA, hsB, dinv_ref, w_ref, b_ref, y_ref) = refs
        dinv = dinv_ref[...]
        zA = (accA[...] + hsA[...]) * dinv
        zB = (accB[...] + hsB[...]) * dinv
        w = w_ref[...]
        y = (jnp.dot(zA, w[:d2], preferred_element_type=jnp.float32)
             + jnp.dot(zB, w[d2:], preferred_element_type=jnp.float32)
             + b_ref[...])
        y_ref[...] = y
        if with_stats:
            i = pl.program_id(0)

            @pl.when(i == 0)
            def _():
                st_ref[...] = jnp.zeros_like(st_ref)

            st_ref[0:1] = st_ref[0:1] + jnp.sum(y, axis=0, keepdims=True)
            st_ref[1:2] = st_ref[1:2] + jnp.sum(y * y, axis=0, keepdims=True)

    out_specs = [pl.BlockSpec((R, dout), lambda i: (i, 0))]
    out_shape = [jax.ShapeDtypeStruct((N, dout), jnp.float32)]
    if with_stats:
        out_specs.append(pl.BlockSpec((8, dout), lambda i: (0, 0)))
        out_shape.append(jax.ShapeDtypeStruct((8, dout), jnp.float32))

    def call(acc, hs, dinv, w, b):
        return pl.pallas_call(
            body,
            grid=(NBLK,),
            in_specs=[
                pl.BlockSpec((R, d2), lambda i: (i, 0)),
                pl.BlockSpec((R, d2), lambda i: (i + NBLK, 0)),
                pl.BlockSpec((R, d2), lambda i: (i, 0)),
                pl.BlockSpec((R, d2), lambda i: (i + NBLK, 0)),
                pl.BlockSpec((R, 1), lambda i: (i, 0)),
                pl.BlockSpec((2 * d2, dout), lambda i: (0, 0)),
                pl.BlockSpec((1, dout), lambda i: (0, 0)),
            ],
            out_specs=out_specs,
            out_shape=out_shape,
        )(acc, acc, hs, hs, dinv, w, b.reshape(1, dout))
    return call


def _k1_l1_body(accA, accB, hs_ref, dinv_ref, w_ref, b_ref, y_ref, st_ref):
    z = (accA[...] + accB[...] + hs_ref[...]) * dinv_ref[...]
    y = jnp.dot(z, w_ref[...], preferred_element_type=jnp.float32) + b_ref[...]
    y_ref[...] = y
    i = pl.program_id(0)

    @pl.when(i == 0)
    def _():
        st_ref[...] = jnp.zeros_like(st_ref)

    st_ref[0:1] = st_ref[0:1] + jnp.sum(y, axis=0, keepdims=True)
    st_ref[1:2] = st_ref[1:2] + jnp.sum(y * y, axis=0, keepdims=True)


def _k1_l1(acc, hs, dinv, w, b):
    return pl.pallas_call(
        _k1_l1_body,
        grid=(NBLK,),
        in_specs=[
            pl.BlockSpec((R, D_IN), lambda i: (i, 0)),
            pl.BlockSpec((R, D_IN), lambda i: (i + NBLK, 0)),
            pl.BlockSpec((R, D_IN), lambda i: (i, 0)),
            pl.BlockSpec((R, 1), lambda i: (i, 0)),
            pl.BlockSpec((D_IN, D_H), lambda i: (0, 0)),
            pl.BlockSpec((1, D_H), lambda i: (0, 0)),
        ],
        out_specs=[
            pl.BlockSpec((R, D_H), lambda i: (i, 0)),
            pl.BlockSpec((8, D_H), lambda i: (0, 0)),
        ],
        out_shape=[
            jax.ShapeDtypeStruct((N, D_H), jnp.float32),
            jax.ShapeDtypeStruct((8, D_H), jnp.float32),
        ],
    )(acc, acc, hs, dinv, w, b.reshape(1, D_H))


_k1_l2 = _make_k1(128, D_H, True)
_k1_l3 = _make_k1(128, D_OUT, False)


def _make_k2(dout):
    half = dout // 2

    def body(y_ref, st_ref, g_ref, be_ref, dinv_ref, o_ref):
        st = st_ref[...]
        mean = st[0:1] / N
        var = st[1:2] / N - mean * mean
        inv = lax.rsqrt(var + 1e-5)
        h = g_ref[...] * (y_ref[...] - mean) * inv + be_ref[...]
        h = jnp.maximum(h, 0.0) * dinv_ref[...]
        o_ref[0] = h[:, :half]
        o_ref[1] = h[:, half:]

    def call(y, st, g, be, dinv):
        return pl.pallas_call(
            body,
            grid=(NBLK,),
            in_specs=[
                pl.BlockSpec((R, dout), lambda i: (i, 0)),
                pl.BlockSpec((8, dout), lambda i: (0, 0)),
                pl.BlockSpec((1, dout), lambda i: (0, 0)),
                pl.BlockSpec((1, dout), lambda i: (0, 0)),
                pl.BlockSpec((R, 1), lambda i: (i, 0)),
            ],
            out_specs=pl.BlockSpec((2, R, half), lambda i: (0, i, 0)),
            out_shape=jax.ShapeDtypeStruct((2, N, half), jnp.float32),
        )(y, st, g.reshape(1, dout), be.reshape(1, dout), dinv)
    return call


_k2_l1 = _make_k2(D_H)
_k2_l2 = _make_k2(D_H)


def _k3_body(b_ref, st_ref):
    i = pl.program_id(0)
    bb = b_ref[...]
    gr = lax.broadcasted_iota(jnp.int32, (1, GPAD), 1)
    eq = (bb == gr).astype(jnp.float32)
    lt = (gr > bb).astype(jnp.float32)

    @pl.when(i == 0)
    def _():
        st_ref[...] = jnp.zeros_like(st_ref)

    st_ref[0:1] = st_ref[0:1] + jnp.sum(eq, axis=0, keepdims=True)
    st_ref[1:2] = st_ref[1:2] + jnp.sum(lt, axis=0, keepdims=True)

    @pl.when(i == NBLK - 1)
    def _():
        st_ref[2:3] = 1.0 / jnp.maximum(st_ref[0:1], 1.0)


def _k3(batch2):
    return pl.pallas_call(
        _k3_body,
        grid=(NBLK,),
        in_specs=[pl.BlockSpec((R, 1), lambda i: (i, 0))],
        out_specs=pl.BlockSpec((8, GPAD), lambda i: (0, 0)),
        out_shape=jax.ShapeDtypeStruct((8, GPAD), jnp.float32),
    )(batch2)


def kernel(x, edge_index, batch, W1, b1, g1, be1, W2, b2, g2, be2, W3, b3):
    src = edge_index[0]
    dst = edge_index[1]

    parts = _deg_call(dst)
    dinv, xs = _k0(parts, x)

    acc1 = _agg_l1(xs, src, dst)
    y1, st1 = _k1_l1(acc1, xs, dinv, W1, b1)
    hs1 = _k2_l1(y1, st1, g1, be1, dinv).reshape(2 * N, D_H // 2)

    acc2 = _agg128(hs1, src, dst)
    y2, st2 = _k1_l2(acc2, hs1, dinv, W2, b2)
    hs2 = _k2_l2(y2, st2, g2, be2, dinv).reshape(2 * N, D_H // 2)

    acc3 = _agg128(hs2, src, dst)
    (h3,) = _k1_l3(acc3, hs2, dinv, W3, b3)

    st = _k3(batch.reshape(N, 1))
    counts = st[0].astype(jnp.int32)
    starts = st[1].astype(jnp.int32)
    rcp = st[2]

    return _pool_call(h3, starts, counts, rcp)
